# Initial kernel scaffold; baseline (speedup 1.0000x reference)
import jax
import jax.numpy as jnp
from jax.experimental import pallas as pl


def kernel(hidden_states):
    raise NotImplementedError("write your pallas kernel here")


# speedup vs baseline: 4.7286x; 4.7286x over previous
import jax
import jax.numpy as jnp
from jax.experimental import pallas as pl

K = 4
_T = 512


def _body(x_ref, halo_ref, o_ref, *, seq_len, chunk):
    j = pl.program_id(1)
    nj = pl.num_programs(1)
    x = x_ref[0]
    halo = halo_ref[0][:K]
    is_last = j == nj - 1
    halo = jnp.where(is_last, jnp.zeros_like(halo), halo)
    ext = jnp.concatenate([x, halo], axis=0)
    total = ext[1:chunk + 1]
    for i in range(2, K + 1):
        total = total + ext[i:chunk + i]
    t_local = jax.lax.broadcasted_iota(jnp.int32, (chunk, 1), 0)
    t_glob = j * chunk + t_local
    lengths = jnp.minimum(K, (seq_len - 1) - t_glob)
    safe = jnp.maximum(lengths, 1).astype(x.dtype)
    mean = total / safe
    o_ref[0] = jnp.where(lengths > 0, mean, jnp.zeros_like(mean))


def kernel(hidden_states):
    B, S, H = hidden_states.shape
    T = _T
    n_chunks = S // T
    halo_blocks = S // 8

    import functools
    body = functools.partial(_body, seq_len=S, chunk=T)
    return pl.pallas_call(
        body,
        grid=(B, n_chunks),
        in_specs=[
            pl.BlockSpec((1, T, H), lambda b, j: (b, j, 0)),
            pl.BlockSpec(
                (1, 8, H),
                lambda b, j: (b, jnp.minimum((j + 1) * (T // 8), halo_blocks - 1), 0),
            ),
        ],
        out_specs=pl.BlockSpec((1, T, H), lambda b, j: (b, j, 0)),
        out_shape=jax.ShapeDtypeStruct((B, S, H), hidden_states.dtype),
    )(hidden_states, hidden_states)
